# baseline (speedup 1.0000x reference)
import functools

import jax
import jax.numpy as jnp
import numpy as np
from jax import lax
from jax.experimental import pallas as pl
from jax.experimental.pallas import tpu as pltpu
from jax.experimental.pallas import tpu_sc as plsc

NUM_TIMESTEPS = 1000
BETA_START = 0.0001
BETA_END = 0.02

_betas = np.linspace(BETA_START ** 0.5, BETA_END ** 0.5, NUM_TIMESTEPS, dtype=np.float64) ** 2
_alphas_cumprod = np.cumprod(1.0 - _betas, axis=0)
_TAB_PAD = 1024
_tab = np.zeros((2 * _TAB_PAD,), dtype=np.float32)
_tab[:NUM_TIMESTEPS] = np.sqrt(_alphas_cumprod)
_tab[_TAB_PAD:_TAB_PAD + NUM_TIMESTEPS] = np.sqrt(1.0 - _alphas_cumprod)

_NC = 2
_NS = 16
_L = 16
_NW = _NC * _NS

_CHUNK = 128


@functools.partial(jax.jit, static_argnames=("batch", "attr"))
def _sc_q_sample(x, n, t, tab, *, batch, attr):
    b_per_w = batch // _NW
    n_chunks = b_per_w // _CHUNK
    mesh = plsc.VectorSubcoreMesh(core_axis_name="c", subcore_axis_name="s")

    @functools.partial(
        pl.kernel,
        mesh=mesh,
        compiler_params=pltpu.CompilerParams(
            needs_layout_passes=False,
            disable_bounds_checks=True,
            disable_semaphore_checks=True,
            skip_device_barrier=True,
        ),
        out_type=jax.ShapeDtypeStruct((batch, attr), jnp.float32),
        scratch_types=[
            pltpu.VMEM((2 * _TAB_PAD,), jnp.float32),
            pltpu.VMEM((b_per_w,), jnp.int32),
            pltpu.VMEM((b_per_w,), jnp.float32),
            pltpu.VMEM((b_per_w,), jnp.float32),
            pltpu.VMEM((2, _CHUNK, attr), jnp.float32),
            pltpu.VMEM((2, _CHUNK, attr), jnp.float32),
            pltpu.VMEM((2, _CHUNK, attr), jnp.float32),
            pltpu.SemaphoreType.DMA,
            pltpu.SemaphoreType.DMA,
            pltpu.SemaphoreType.DMA,
            pltpu.SemaphoreType.DMA,
        ],
    )
    def k(x_hbm, n_hbm, t_hbm, tab_hbm, o_hbm,
          tab_v, t_v, c1_v, c2_v, x_v, n_v, o_v,
          in_sem, in_sem2, out_sem, stage_sem):
        wid = lax.axis_index("s") * _NC + lax.axis_index("c")
        base = wid * b_per_w

        def start_in(kk, slot):
            row0 = base + kk * _CHUNK
            cp_x = pltpu.make_async_copy(
                x_hbm.at[pl.ds(row0, _CHUNK), :], x_v.at[slot], in_sem)
            cp_n = pltpu.make_async_copy(
                n_hbm.at[pl.ds(row0, _CHUNK), :], n_v.at[slot], in_sem2)
            cp_x.start()
            cp_n.start()
            return cp_x, cp_n

        def compute(kk, slot):
            crow0 = kk * _CHUNK

            @plsc.parallel_loop(0, _CHUNK // _L, 1, unroll=1)
            def gbody2(g):
                c1g = c1_v[pl.ds(crow0 + g * _L, _L)]
                c2g = c2_v[pl.ds(crow0 + g * _L, _L)]
                r0 = g * _L
                for r in range(_L):
                    c1s = c1g[r]
                    c2s = c2g[r]
                    for s in range(attr // _L):
                        sl = pl.ds(s * _L, _L)
                        o_v[slot, r0 + r, sl] = (c1s * x_v[slot, r0 + r, sl]
                                                 + c2s * n_v[slot, r0 + r, sl])

        def start_out(kk, slot):
            row0 = base + kk * _CHUNK
            cp = pltpu.make_async_copy(
                o_v.at[slot], o_hbm.at[pl.ds(row0, _CHUNK), :], out_sem)
            cp.start()
            return cp

        pending_out = [None, None]
        cps = [None] * n_chunks
        cps[0] = start_in(0, 0)
        if n_chunks > 1:
            cps[1] = start_in(1, 1)

        cp_tab = pltpu.make_async_copy(tab_hbm, tab_v, stage_sem)
        cp_t = pltpu.make_async_copy(t_hbm.at[pl.ds(base, b_per_w)], t_v, stage_sem)
        cp_tab.start()
        cp_t.start()
        cp_tab.wait()
        cp_t.wait()

        def gbody(i, carry):
            idx = t_v[pl.ds(i * _L, _L)]
            c1_v[pl.ds(i * _L, _L)] = plsc.load_gather(tab_v, [idx])
            c2_v[pl.ds(i * _L, _L)] = plsc.load_gather(tab_v, [idx + _TAB_PAD])
            return carry

        lax.fori_loop(0, b_per_w // _L, gbody, 0)

        for kk in range(n_chunks):
            slot = kk % 2
            cps[kk][0].wait()
            cps[kk][1].wait()
            if pending_out[slot] is not None:
                pending_out[slot].wait()
            compute(kk, slot)
            pending_out[slot] = start_out(kk, slot)
            if kk + 2 < n_chunks:
                cps[kk + 2] = start_in(kk + 2, slot)
        for cp in pending_out:
            if cp is not None:
                cp.wait()

    return k(x, n, t, tab)


def kernel(x_start, noise, t):
    batch, attr = x_start.shape
    return _sc_q_sample(x_start, noise, t.astype(jnp.int32),
                        jnp.asarray(_tab), batch=batch, attr=attr)


# speedup vs baseline: 1.0123x; 1.0123x over previous
import functools

import jax
import jax.numpy as jnp
import numpy as np
from jax import lax
from jax.experimental import pallas as pl
from jax.experimental.pallas import tpu as pltpu
from jax.experimental.pallas import tpu_sc as plsc

NUM_TIMESTEPS = 1000
BETA_START = 0.0001
BETA_END = 0.02

_betas = np.linspace(BETA_START ** 0.5, BETA_END ** 0.5, NUM_TIMESTEPS, dtype=np.float64) ** 2
_alphas_cumprod = np.cumprod(1.0 - _betas, axis=0)
_TAB_PAD = 1024
_tab = np.zeros((2 * _TAB_PAD,), dtype=np.float32)
_tab[:NUM_TIMESTEPS] = np.sqrt(_alphas_cumprod)
_tab[_TAB_PAD:_TAB_PAD + NUM_TIMESTEPS] = np.sqrt(1.0 - _alphas_cumprod)

_NC = 2
_NS = 16
_L = 16
_NW = _NC * _NS

_CHUNK = 128


@functools.partial(jax.jit, static_argnames=("batch", "attr"))
def _sc_q_sample(x, n, t, tab, *, batch, attr):
    b_per_w = batch // _NW
    n_chunks = b_per_w // _CHUNK
    mesh = plsc.VectorSubcoreMesh(core_axis_name="c", subcore_axis_name="s")

    @functools.partial(
        pl.kernel,
        mesh=mesh,
        compiler_params=pltpu.CompilerParams(
            needs_layout_passes=False,
            disable_bounds_checks=True,
            disable_semaphore_checks=True,
            skip_device_barrier=True,
        ),
        out_type=jax.ShapeDtypeStruct((batch, attr), jnp.float32),
        scratch_types=[
            pltpu.VMEM((2 * _TAB_PAD,), jnp.float32),
            pltpu.VMEM((b_per_w,), jnp.int32),
            pltpu.VMEM((b_per_w,), jnp.float32),
            pltpu.VMEM((b_per_w,), jnp.float32),
            pltpu.VMEM((2, _CHUNK, attr), jnp.float32),
            pltpu.VMEM((2, _CHUNK, attr), jnp.float32),
            pltpu.VMEM((2, _CHUNK, attr), jnp.float32),
            pltpu.SemaphoreType.DMA,
            pltpu.SemaphoreType.DMA,
            pltpu.SemaphoreType.DMA,
            pltpu.SemaphoreType.DMA,
        ],
    )
    def k(x_hbm, n_hbm, t_hbm, tab_hbm, o_hbm,
          tab_v, t_v, c1_v, c2_v, x_v, n_v, o_v,
          in_sem, in_sem2, out_sem, stage_sem):
        wid = lax.axis_index("s") * _NC + lax.axis_index("c")
        base = wid * b_per_w

        def start_in(kk, slot):
            row0 = base + kk * _CHUNK
            cp_x = pltpu.make_async_copy(
                x_hbm.at[pl.ds(row0, _CHUNK), :], x_v.at[slot], in_sem)
            cp_n = pltpu.make_async_copy(
                n_hbm.at[pl.ds(row0, _CHUNK), :], n_v.at[slot], in_sem2)
            cp_x.start()
            cp_n.start()
            return cp_x, cp_n

        def compute(kk, slot):
            crow0 = kk * _CHUNK

            @plsc.parallel_loop(0, _CHUNK // _L, 1, unroll=1)
            def gbody2(g):
                c1g = c1_v[pl.ds(crow0 + g * _L, _L)]
                c2g = c2_v[pl.ds(crow0 + g * _L, _L)]
                r0 = g * _L
                for r in range(_L):
                    c1s = c1g[r]
                    c2s = c2g[r]
                    for s in range(attr // _L):
                        sl = pl.ds(s * _L, _L)
                        o_v[slot, r0 + r, sl] = (c1s * x_v[slot, r0 + r, sl]
                                                 + c2s * n_v[slot, r0 + r, sl])

        def start_out(kk, slot):
            row0 = base + kk * _CHUNK
            cp = pltpu.make_async_copy(
                o_v.at[slot], o_hbm.at[pl.ds(row0, _CHUNK), :], out_sem)
            cp.start()
            return cp

        pending_out = [None, None]
        cps = start_in(0, 0)

        cp_tab = pltpu.make_async_copy(tab_hbm, tab_v, stage_sem)
        cp_t = pltpu.make_async_copy(t_hbm.at[pl.ds(base, b_per_w)], t_v, stage_sem)
        cp_tab.start()
        cp_t.start()
        cp_tab.wait()
        cp_t.wait()

        def gbody(i, carry):
            idx = t_v[pl.ds(i * _L, _L)]
            c1_v[pl.ds(i * _L, _L)] = plsc.load_gather(tab_v, [idx])
            c2_v[pl.ds(i * _L, _L)] = plsc.load_gather(tab_v, [idx + _TAB_PAD])
            return carry

        lax.fori_loop(0, b_per_w // _L, gbody, 0)

        for kk in range(n_chunks):
            slot = kk % 2
            nslot = 1 - slot
            if kk + 1 < n_chunks:
                nxt = start_in(kk + 1, nslot)
            cps[0].wait()
            cps[1].wait()
            if pending_out[slot] is not None:
                pending_out[slot].wait()
            compute(kk, slot)
            pending_out[slot] = start_out(kk, slot)
            if kk + 1 < n_chunks:
                cps = nxt
        for cp in pending_out:
            if cp is not None:
                cp.wait()

    return k(x, n, t, tab)


def kernel(x_start, noise, t):
    batch, attr = x_start.shape
    return _sc_q_sample(x_start, noise, t.astype(jnp.int32),
                        jnp.asarray(_tab), batch=batch, attr=attr)
